# baseline (speedup 1.0000x reference)
import functools
import jax
import jax.numpy as jnp
from jax import lax
from jax.experimental import pallas as pl
from jax.experimental.pallas import tpu as pltpu
from jax.experimental.pallas import tpu_sc as plsc

N = 10000
NACC = 10016
E = 320000
ET = E + N
NW = 32
BLK = 128
NBLK = 82
NB0 = 106
NB1 = 2 * NBLK - NB0
ET_PAD = NW * NBLK * BLK
NEG = -1e30
BN = 400
GRID = N // BN
F32 = jnp.float32


def _tc1_body(x_ref, w1t_ref, a1_ref, hext_ref, adst_ref):
    h = jnp.dot(x_ref[...], w1t_ref[...], preferred_element_type=F32)
    al = jnp.dot(h, a1_ref[...], preferred_element_type=F32)
    z8 = jnp.zeros((h.shape[0], 8), F32)
    hext_ref[...] = jnp.concatenate([h, al[:, 0:8]], axis=1)
    adst_ref[...] = jnp.concatenate([z8, al[:, 8:16]], axis=1)


def _tc2_body(p_ref, b1_ref, r_ref, w2t_ref, a2_ref, hext_ref, adst_ref):
    p = p_ref[0] + p_ref[1]
    num = p[:, 0:64]
    den = jnp.dot(p[:, 64:72], r_ref[...], preferred_element_type=F32)
    h1 = num / (den + 1e-16) + b1_ref[...]
    h2 = jnp.dot(h1, w2t_ref[...], preferred_element_type=F32)
    al2 = jnp.dot(h2, a2_ref[...], preferred_element_type=F32)
    z7 = jnp.zeros((h2.shape[0], 7), F32)
    z8 = jnp.zeros((h2.shape[0], 8), F32)
    hext_ref[...] = jnp.concatenate([h2, al2[:, 0:1], z7], axis=1)
    adst_ref[...] = jnp.concatenate([z8, al2[:, 1:2], z7], axis=1)


def _tc3_body(p_ref, b2_ref, out_ref):
    p = p_ref[0] + p_ref[1]
    out_ref[...] = p[:, 0:128] / (p[:, 128:129] + 1e-16) + b2_ref[...]


_tc1 = pl.pallas_call(
    _tc1_body,
    grid=(GRID,),
    in_specs=[
        pl.BlockSpec((BN, 128), lambda i: (i, 0)),
        pl.BlockSpec((128, 64), lambda i: (0, 0)),
        pl.BlockSpec((64, 16), lambda i: (0, 0)),
    ],
    out_specs=[
        pl.BlockSpec((BN, 72), lambda i: (i, 0)),
        pl.BlockSpec((BN, 16), lambda i: (i, 0)),
    ],
    out_shape=[
        jax.ShapeDtypeStruct((N, 72), F32),
        jax.ShapeDtypeStruct((N, 16), F32),
    ],
)

_tc2 = pl.pallas_call(
    _tc2_body,
    grid=(GRID,),
    in_specs=[
        pl.BlockSpec((2, BN, 72), lambda i: (0, i, 0)),
        pl.BlockSpec((1, 64), lambda i: (0, 0)),
        pl.BlockSpec((8, 64), lambda i: (0, 0)),
        pl.BlockSpec((64, 128), lambda i: (0, 0)),
        pl.BlockSpec((128, 16), lambda i: (0, 0)),
    ],
    out_specs=[
        pl.BlockSpec((BN, 136), lambda i: (i, 0)),
        pl.BlockSpec((BN, 16), lambda i: (i, 0)),
    ],
    out_shape=[
        jax.ShapeDtypeStruct((N, 136), F32),
        jax.ShapeDtypeStruct((N, 16), F32),
    ],
)

_tc3 = pl.pallas_call(
    _tc3_body,
    grid=(GRID,),
    in_specs=[
        pl.BlockSpec((2, BN, 136), lambda i: (0, i, 0)),
        pl.BlockSpec((1, 128), lambda i: (0, 0)),
    ],
    out_specs=pl.BlockSpec((BN, 128), lambda i: (i, 0)),
    out_shape=jax.ShapeDtypeStruct((N, 128), F32),
)


def _dyng(v, idx):
    dnums = lax.GatherDimensionNumbers(
        offset_dims=(), collapsed_slice_dims=(0,), start_index_map=(0,))
    return lax.gather(v, idx[:, None], dnums, (1,),
                      mode=lax.GatherScatterMode.PROMISE_IN_BOUNDS)


def _make_sc_edge(heads, row, nb0=NBLK, nb1=NBLK):
    mesh = plsc.VectorSubcoreMesh(core_axis_name="c", subcore_axis_name="s",
                                  num_cores=2, num_subcores=16)
    d = row - 8
    nch = d // 16
    tl = row - 16

    @functools.partial(
        pl.kernel,
        out_type=jax.ShapeDtypeStruct((2, NACC, row), F32),
        mesh=mesh,
        scratch_types=(
            [pltpu.VMEM((2, 2, BLK), jnp.int32)] * 2
            + [pltpu.VMEM((BLK, row), F32)] * 2
            + [pltpu.VMEM((BLK, 16), F32)] * 2
            + [pltpu.VMEM_SHARED((NACC, row), F32)]
            + [pltpu.SemaphoreType.DMA] * 6
        ),
        compiler_params=pltpu.CompilerParams(use_tc_tiling_on_sc=False),
    )
    def k(hext_hbm, adst_hbm, src_hbm, dst_hbm, out_hbm,
          srcv, dstv, h0, h1, a0, a1, acc, sg0, sg1, sa0, sa1, si, sd):
        hb = [h0, h1]
        ab = [a0, a1]
        sg = [sg0, sg1]
        sa = [sa0, sa1]
        cid = lax.axis_index("c")
        sid = lax.axis_index("s")

        nbc = jnp.where(cid == 0, nb0, nb1)
        nsc = nbc // 2
        wbase = cid * (16 * nb0) + sid * nbc
        iota = lax.iota(jnp.int32, 16)
        hi3 = iota >> 3
        zeros = jnp.zeros((16,), F32)

        pltpu.sync_copy(src_hbm.at[pl.ds(wbase, 2)], srcv.at[0])
        pltpu.sync_copy(dst_hbm.at[pl.ds(wbase, 2)], dstv.at[0])

        def zb(e, c):
            for cc in range(nch):
                h0[e, pl.ds(cc * 16, 16)] = zeros
            h0[e, pl.ds(tl, 16)] = zeros
            return c
        lax.fori_loop(0, BLK, zb, 0)
        rpt = NACC // 16
        rbase = sid * rpt
        for j in range(rpt // BLK):
            pltpu.sync_copy(h0, acc.at[pl.ds(rbase + j * BLK, BLK)])
        rtail = rpt % BLK
        if rtail:
            pltpu.sync_copy(h0.at[pl.ds(0, rtail)],
                            acc.at[pl.ds(rbase + (rpt // BLK) * BLK, rtail)])

        pltpu.async_copy(hext_hbm.at[srcv.at[0, 0]], h0, sg0)
        pltpu.async_copy(adst_hbm.at[dstv.at[0, 0]], a0, sa0)

        plsc.subcore_barrier()

        def compute(b):
            hr = hb[b]
            ar = ab[b]

            def msg_body(e, c):
                t0 = hr[e, pl.ds(tl, 16)]
                av = ar[e, pl.ds(0, 16)]
                s = t0 + av
                s = jnp.where(s >= 0.0, s, s * F32(0.2))
                ea = jnp.exp(s)
                for cc in range(nch):
                    if heads == 8:
                        eam = _dyng(ea, 8 + cc * 2 + hi3)
                    else:
                        eam = _dyng(ea, iota * 0 + 8)
                    hr[e, pl.ds(cc * 16, 16)] = (
                        hr[e, pl.ds(cc * 16, 16)] * eam)
                cur = hr[e, pl.ds(tl, 16)]
                if heads == 8:
                    tail = jnp.where(iota < 8, cur, ea)
                else:
                    tail = jnp.where(iota < 8, cur,
                                     jnp.where(iota == 8, ea, zeros))
                hr[e, pl.ds(tl, 16)] = tail
                return c

            lax.fori_loop(0, BLK, msg_body, 0, unroll=2)

        def superstep(ks, carry):
            p = ks & 1
            pn = 1 - p
            nxt = wbase + (ks + 1) * 2

            @pl.when(ks + 1 < nsc)
            def _():
                pltpu.async_copy(src_hbm.at[pl.ds(nxt, 2)], srcv.at[pn],
                                 si)
                pltpu.async_copy(dst_hbm.at[pl.ds(nxt, 2)], dstv.at[pn],
                                 sd)

            for j in range(2):
                b = j
                pltpu.make_async_copy(hext_hbm.at[srcv.at[p, j]], hb[b],
                                      sg[b]).wait()
                pltpu.make_async_copy(adst_hbm.at[dstv.at[p, j]], ab[b],
                                      sa[b]).wait()
                if j == 0:
                    pltpu.async_copy(hext_hbm.at[srcv.at[p, 1]], h1, sg1)
                    pltpu.async_copy(adst_hbm.at[dstv.at[p, 1]], a1, sa1)
                else:
                    @pl.when(ks + 1 < nsc)
                    def _():
                        pltpu.make_async_copy(
                            src_hbm.at[pl.ds(nxt, 2)], srcv.at[pn],
                            si).wait()
                        pltpu.make_async_copy(
                            dst_hbm.at[pl.ds(nxt, 2)], dstv.at[pn],
                            sd).wait()
                        pltpu.async_copy(hext_hbm.at[srcv.at[pn, 0]], h0,
                                         sg0)
                        pltpu.async_copy(adst_hbm.at[dstv.at[pn, 0]], a0,
                                         sa0)
                compute(b)
                pltpu.sync_copy(hb[b], acc.at[dstv.at[p, j]], add=True)
            return carry

        lax.fori_loop(0, nsc, superstep, 0)
        plsc.subcore_barrier()

        @pl.when(sid == 0)
        def _():
            pltpu.sync_copy(acc, out_hbm.at[cid])

    return k


_make_sc_edge = functools.lru_cache(maxsize=None)(_make_sc_edge)


def kernel(x, edge_index, W1, att_src1, att_dst1, b1, W2, att_src2, att_dst2, b2):
    eye8 = jnp.eye(8, dtype=F32)
    a1s = jnp.einsum('hj,hk->hjk', att_src1, eye8).reshape(64, 8)
    a1d = jnp.einsum('hj,hk->hjk', att_dst1, eye8).reshape(64, 8)
    a1 = jnp.concatenate([a1s, a1d], axis=1)
    r8 = jnp.kron(eye8, jnp.ones((1, 8), F32))
    a2 = jnp.concatenate([att_src2.T, att_dst2.T,
                          jnp.zeros((128, 14), F32)], axis=1)
    w1t = W1.T
    w2t = W2.T
    b1r = b1.reshape(1, 64)
    b2r = b2.reshape(1, 128)

    loops = jnp.arange(N, dtype=jnp.int32)
    pad = jnp.full((ET_PAD - ET,), N, jnp.int32)
    srcp = jnp.concatenate([edge_index[0].astype(jnp.int32), loops, pad])
    dstp = jnp.concatenate([edge_index[1].astype(jnp.int32), loops, pad])
    srcp = srcp.reshape(-1, BLK)
    dstp = dstp.reshape(-1, BLK)

    hext1, adst1 = _tc1(x, w1t, a1)
    prow1 = jnp.concatenate([jnp.zeros((1, 64), F32),
                             jnp.full((1, 8), NEG, F32)], axis=1)
    arow1 = jnp.concatenate([jnp.zeros((1, 8), F32),
                             jnp.full((1, 8), NEG, F32)], axis=1)
    hext1 = jnp.concatenate([hext1, prow1], axis=0)
    adst1 = jnp.concatenate([adst1, arow1], axis=0)
    part1 = _make_sc_edge(8, 72, NB0, NB1)(hext1, adst1, srcp, dstp)

    hext2, adst2 = _tc2(part1, b1r, r8, w2t, a2)
    prow2 = jnp.concatenate([jnp.zeros((1, 128), F32),
                             jnp.full((1, 1), NEG, F32),
                             jnp.zeros((1, 7), F32)], axis=1)
    arow2 = jnp.concatenate([jnp.zeros((1, 8), F32),
                             jnp.full((1, 1), NEG, F32),
                             jnp.zeros((1, 7), F32)], axis=1)
    hext2 = jnp.concatenate([hext2, prow2], axis=0)
    adst2 = jnp.concatenate([adst2, arow2], axis=0)
    part2 = _make_sc_edge(1, 136, NB0, NB1)(hext2, adst2, srcp, dstp)

    return _tc3(part2, b2r)


# speedup vs baseline: 1.0312x; 1.0312x over previous
import functools
import jax
import jax.numpy as jnp
from jax import lax
from jax.experimental import pallas as pl
from jax.experimental.pallas import tpu as pltpu
from jax.experimental.pallas import tpu_sc as plsc

N = 10000
NACC = 10016
E = 320000
ET = E + N
NW = 32
BLK = 128
NBLK = 82
NB0 = 106
NB1 = 2 * NBLK - NB0
ET_PAD = NW * NBLK * BLK
NEG = -1e30
BN = 400
GRID = N // BN
F32 = jnp.float32


def _tc1_body(x_ref, w1t_ref, a1_ref, hext_ref, adst_ref):
    h = jnp.dot(x_ref[...], w1t_ref[...], preferred_element_type=F32)
    al = jnp.dot(h, a1_ref[...], preferred_element_type=F32)
    z8 = jnp.zeros((h.shape[0], 8), F32)
    hext_ref[...] = jnp.concatenate([h, al[:, 0:8]], axis=1)
    adst_ref[...] = jnp.concatenate([z8, al[:, 8:16]], axis=1)


def _tc2_body(p_ref, b1_ref, r_ref, w2t_ref, a2_ref, hext_ref, adst_ref):
    p = p_ref[0] + p_ref[1]
    num = p[:, 0:64]
    den = jnp.dot(p[:, 64:72], r_ref[...], preferred_element_type=F32)
    h1 = num / (den + 1e-16) + b1_ref[...]
    h2 = jnp.dot(h1, w2t_ref[...], preferred_element_type=F32)
    al2 = jnp.dot(h2, a2_ref[...], preferred_element_type=F32)
    z7 = jnp.zeros((h2.shape[0], 7), F32)
    z8 = jnp.zeros((h2.shape[0], 8), F32)
    hext_ref[...] = jnp.concatenate([h2, al2[:, 0:1], z7], axis=1)
    adst_ref[...] = jnp.concatenate([z8, al2[:, 1:2], z7], axis=1)


def _tc3_body(p_ref, b2_ref, out_ref):
    p = p_ref[0] + p_ref[1]
    out_ref[...] = p[:, 0:128] / (p[:, 128:129] + 1e-16) + b2_ref[...]


_tc1 = pl.pallas_call(
    _tc1_body,
    grid=(GRID,),
    in_specs=[
        pl.BlockSpec((BN, 128), lambda i: (i, 0)),
        pl.BlockSpec((128, 64), lambda i: (0, 0)),
        pl.BlockSpec((64, 16), lambda i: (0, 0)),
    ],
    out_specs=[
        pl.BlockSpec((BN, 72), lambda i: (i, 0)),
        pl.BlockSpec((BN, 16), lambda i: (i, 0)),
    ],
    out_shape=[
        jax.ShapeDtypeStruct((NACC, 72), F32),
        jax.ShapeDtypeStruct((NACC, 16), F32),
    ],
)

_tc2 = pl.pallas_call(
    _tc2_body,
    grid=(GRID,),
    in_specs=[
        pl.BlockSpec((2, BN, 72), lambda i: (0, i, 0)),
        pl.BlockSpec((1, 64), lambda i: (0, 0)),
        pl.BlockSpec((8, 64), lambda i: (0, 0)),
        pl.BlockSpec((64, 128), lambda i: (0, 0)),
        pl.BlockSpec((128, 16), lambda i: (0, 0)),
    ],
    out_specs=[
        pl.BlockSpec((BN, 136), lambda i: (i, 0)),
        pl.BlockSpec((BN, 16), lambda i: (i, 0)),
    ],
    out_shape=[
        jax.ShapeDtypeStruct((NACC, 136), F32),
        jax.ShapeDtypeStruct((NACC, 16), F32),
    ],
)

_tc3 = pl.pallas_call(
    _tc3_body,
    grid=(GRID,),
    in_specs=[
        pl.BlockSpec((2, BN, 136), lambda i: (0, i, 0)),
        pl.BlockSpec((1, 128), lambda i: (0, 0)),
    ],
    out_specs=pl.BlockSpec((BN, 128), lambda i: (i, 0)),
    out_shape=jax.ShapeDtypeStruct((N, 128), F32),
)


def _dyng(v, idx):
    dnums = lax.GatherDimensionNumbers(
        offset_dims=(), collapsed_slice_dims=(0,), start_index_map=(0,))
    return lax.gather(v, idx[:, None], dnums, (1,),
                      mode=lax.GatherScatterMode.PROMISE_IN_BOUNDS)


def _make_sc_edge(heads, row, nb0=NBLK, nb1=NBLK):
    mesh = plsc.VectorSubcoreMesh(core_axis_name="c", subcore_axis_name="s",
                                  num_cores=2, num_subcores=16)
    d = row - 8
    nch = d // 16
    tl = row - 16

    @functools.partial(
        pl.kernel,
        out_type=jax.ShapeDtypeStruct((2, NACC, row), F32),
        mesh=mesh,
        scratch_types=(
            [pltpu.VMEM((2, 2, BLK), jnp.int32)] * 2
            + [pltpu.VMEM((BLK, row), F32)] * 2
            + [pltpu.VMEM((BLK, 16), F32)] * 2
            + [pltpu.VMEM_SHARED((NACC, row), F32)]
            + [pltpu.SemaphoreType.DMA] * 6
        ),
        compiler_params=pltpu.CompilerParams(use_tc_tiling_on_sc=False),
    )
    def k(hext_hbm, adst_hbm, src_hbm, dst_hbm, out_hbm,
          srcv, dstv, h0, h1, a0, a1, acc, sg0, sg1, sa0, sa1, si, sd):
        hb = [h0, h1]
        ab = [a0, a1]
        sg = [sg0, sg1]
        sa = [sa0, sa1]
        cid = lax.axis_index("c")
        sid = lax.axis_index("s")

        nbc = jnp.where(cid == 0, nb0, nb1)
        nsc = nbc // 2
        wbase = cid * (16 * nb0) + sid * nbc
        iota = lax.iota(jnp.int32, 16)
        hi3 = iota >> 3
        zeros = jnp.zeros((16,), F32)

        pltpu.sync_copy(src_hbm.at[pl.ds(wbase, 2)], srcv.at[0])
        pltpu.sync_copy(dst_hbm.at[pl.ds(wbase, 2)], dstv.at[0])

        def zb(e, c):
            for cc in range(nch):
                h0[e, pl.ds(cc * 16, 16)] = zeros
            h0[e, pl.ds(tl, 16)] = zeros
            return c
        lax.fori_loop(0, BLK, zb, 0)
        rpt = NACC // 16
        rbase = sid * rpt
        for j in range(rpt // BLK):
            pltpu.sync_copy(h0, acc.at[pl.ds(rbase + j * BLK, BLK)])
        rtail = rpt % BLK
        if rtail:
            pltpu.sync_copy(h0.at[pl.ds(0, rtail)],
                            acc.at[pl.ds(rbase + (rpt // BLK) * BLK, rtail)])

        pltpu.async_copy(hext_hbm.at[srcv.at[0, 0]], h0, sg0)
        pltpu.async_copy(adst_hbm.at[dstv.at[0, 0]], a0, sa0)

        plsc.subcore_barrier()

        def compute(b):
            hr = hb[b]
            ar = ab[b]

            def msg_body(e, c):
                t0 = hr[e, pl.ds(tl, 16)]
                av = ar[e, pl.ds(0, 16)]
                s = t0 + av
                s = jnp.where(s >= 0.0, s, s * F32(0.2))
                ea = jnp.exp(s)
                for cc in range(nch):
                    if heads == 8:
                        eam = _dyng(ea, 8 + cc * 2 + hi3)
                    else:
                        eam = _dyng(ea, iota * 0 + 8)
                    hr[e, pl.ds(cc * 16, 16)] = (
                        hr[e, pl.ds(cc * 16, 16)] * eam)
                cur = hr[e, pl.ds(tl, 16)]
                if heads == 8:
                    tail = jnp.where(iota < 8, cur, ea)
                else:
                    tail = jnp.where(iota < 8, cur,
                                     jnp.where(iota == 8, ea, zeros))
                hr[e, pl.ds(tl, 16)] = tail
                return c

            lax.fori_loop(0, BLK, msg_body, 0, unroll=4)

        def superstep(ks, carry):
            p = ks & 1
            pn = 1 - p
            nxt = wbase + (ks + 1) * 2

            @pl.when(ks + 1 < nsc)
            def _():
                pltpu.async_copy(src_hbm.at[pl.ds(nxt, 2)], srcv.at[pn],
                                 si)
                pltpu.async_copy(dst_hbm.at[pl.ds(nxt, 2)], dstv.at[pn],
                                 sd)

            for j in range(2):
                b = j
                pltpu.make_async_copy(hext_hbm.at[srcv.at[p, j]], hb[b],
                                      sg[b]).wait()
                pltpu.make_async_copy(adst_hbm.at[dstv.at[p, j]], ab[b],
                                      sa[b]).wait()
                if j == 0:
                    pltpu.async_copy(hext_hbm.at[srcv.at[p, 1]], h1, sg1)
                    pltpu.async_copy(adst_hbm.at[dstv.at[p, 1]], a1, sa1)
                else:
                    @pl.when(ks + 1 < nsc)
                    def _():
                        pltpu.make_async_copy(
                            src_hbm.at[pl.ds(nxt, 2)], srcv.at[pn],
                            si).wait()
                        pltpu.make_async_copy(
                            dst_hbm.at[pl.ds(nxt, 2)], dstv.at[pn],
                            sd).wait()
                        pltpu.async_copy(hext_hbm.at[srcv.at[pn, 0]], h0,
                                         sg0)
                        pltpu.async_copy(adst_hbm.at[dstv.at[pn, 0]], a0,
                                         sa0)
                compute(b)
                pltpu.sync_copy(hb[b], acc.at[dstv.at[p, j]], add=True)
            return carry

        lax.fori_loop(0, nsc, superstep, 0)
        plsc.subcore_barrier()

        @pl.when(sid == 0)
        def _():
            pltpu.sync_copy(acc, out_hbm.at[cid])

    return k


_make_sc_edge = functools.lru_cache(maxsize=None)(_make_sc_edge)


def kernel(x, edge_index, W1, att_src1, att_dst1, b1, W2, att_src2, att_dst2, b2):
    eye8 = jnp.eye(8, dtype=F32)
    a1s = jnp.einsum('hj,hk->hjk', att_src1, eye8).reshape(64, 8)
    a1d = jnp.einsum('hj,hk->hjk', att_dst1, eye8).reshape(64, 8)
    a1 = jnp.concatenate([a1s, a1d], axis=1)
    r8 = jnp.kron(eye8, jnp.ones((1, 8), F32))
    a2 = jnp.concatenate([att_src2.T, att_dst2.T,
                          jnp.zeros((128, 14), F32)], axis=1)
    w1t = W1.T
    w2t = W2.T
    b1r = b1.reshape(1, 64)
    b2r = b2.reshape(1, 128)

    loops = jnp.arange(N, dtype=jnp.int32)
    pad = jnp.full((ET_PAD - ET,), N, jnp.int32)
    srcp = jnp.concatenate([edge_index[0].astype(jnp.int32), loops, pad])
    dstp = jnp.concatenate([edge_index[1].astype(jnp.int32), loops, pad])
    srcp = srcp.reshape(-1, BLK)
    dstp = dstp.reshape(-1, BLK)

    hext1, adst1 = _tc1(x, w1t, a1)
    part1 = _make_sc_edge(8, 72, NB0, NB1)(hext1, adst1, srcp, dstp)

    hext2, adst2 = _tc2(part1, b1r, r8, w2t, a2)
    part2 = _make_sc_edge(1, 136, NB0, NB1)(hext2, adst2, srcp, dstp)

    return _tc3(part2, b2r)
